# Initial kernel scaffold; baseline (speedup 1.0000x reference)
import jax
import jax.numpy as jnp
from jax.experimental import pallas as pl


def kernel(emb_vision, emb_nlp, emb_security, emb_medical, hard_sample_weights, W1, b1, W2, b2, domain_weights, domain_ids, pos_rand):
    raise NotImplementedError("write your pallas kernel here")


# speedup vs baseline: 8.3859x; 8.3859x over previous
import functools

import jax
import jax.numpy as jnp
from jax import lax
from jax.experimental import pallas as pl

B = 512
D = 256
N = 4 * B
NUM_NEG = 128
ALPHA = 0.5
TILE = 256
GRID = N // TILE
BISECT_ITERS = 26
MASK_FILL = -5.0


def _loss_kernel(emb_ref, w1_ref, b1_ref, w2_ref, b2_ref, dw_ref, hw_ref,
                 pos_ref, loss_ref, reg_ref):
    i = pl.program_id(0)

    emb_full = emb_ref[...]
    nrm = jnp.sqrt(jnp.sum(emb_full * emb_full, axis=1, keepdims=True))
    emb_n = emb_full / jnp.maximum(nrm, 1e-12)

    row0 = i * TILE
    tile_raw = emb_ref[pl.ds(row0, TILE), :]
    tile_nrm = jnp.sqrt(jnp.sum(tile_raw * tile_raw, axis=1, keepdims=True))
    tile_n = tile_raw / jnp.maximum(tile_nrm, 1e-12)

    sim = lax.dot_general(tile_n, emb_n, (((1,), (1,)), ((), ())),
                          preferred_element_type=jnp.float32)

    h = jnp.maximum(jnp.dot(tile_raw, w1_ref[...],
                            preferred_element_type=jnp.float32)
                    + b1_ref[...], 0.0)
    tlin = jnp.dot(h, w2_ref[...], preferred_element_type=jnp.float32) \
        + b2_ref[...]
    temps = 0.01 + 0.99 * jax.nn.sigmoid(tlin)
    inv_t = 1.0 / temps

    local = row0 % B + lax.broadcasted_iota(jnp.int32, (TILE, 1), 0)
    pr = pos_ref[pl.ds(row0, TILE), :]
    pos_local = pr + (pr >= local).astype(jnp.int32)
    pos_idx = (row0 // B) * B + pos_local

    col = lax.broadcasted_iota(jnp.int32, (TILE, N), 1)
    pos_sim = jnp.sum(jnp.where(col == pos_idx, sim, 0.0), axis=1,
                      keepdims=True)

    dom = row0 // B
    masked = jnp.where((col // B) == dom, MASK_FILL, sim)

    row_max = jnp.max(masked, axis=1, keepdims=True)
    m = jnp.maximum(pos_sim, row_max)

    lo0 = jnp.full((TILE, 1), -1.01, jnp.float32)
    hi0 = row_max

    def body(_, carry):
        lo, hi = carry
        mid = 0.5 * (lo + hi)
        cnt = jnp.sum((masked > mid).astype(jnp.float32), axis=1,
                      keepdims=True)
        ge = cnt >= float(NUM_NEG)
        return jnp.where(ge, mid, lo), jnp.where(ge, hi, mid)

    lo, hi = lax.fori_loop(0, BISECT_ITERS, body, (lo0, hi0))

    cnt_hi = jnp.sum((masked > hi).astype(jnp.float32), axis=1, keepdims=True)
    exps = jnp.where(masked > hi, jnp.exp((masked - m) * inv_t), 0.0)
    sum_top = jnp.sum(exps, axis=1, keepdims=True) \
        + (float(NUM_NEG) - cnt_hi) * jnp.exp((lo - m) * inv_t)
    total = jnp.exp((pos_sim - m) * inv_t) + sum_top
    losses = ((m - pos_sim) * inv_t + jnp.log(total)) * hw_ref[pl.ds(row0, TILE), :]
    part = jnp.sum(losses).reshape(1, 1)

    @pl.when(i == 0)
    def _():
        loss_ref[...] = jnp.zeros((1, 1), jnp.float32)
        cent = jnp.mean(emb_full.reshape(4, B, D), axis=1)
        reg = jnp.zeros((1, 1), jnp.float32)
        for a in range(4):
            for b in range(a + 1, 4):
                dvec = cent[a] - cent[b]
                reg = reg + dw_ref[a, b] * jnp.sqrt(jnp.sum(dvec * dvec))
        reg_ref[...] = reg / 6.0

    loss_ref[...] += part


@functools.partial(jax.jit, static_argnames=("interpret",))
def _run(all_emb, w1, b1, w2, b2, dw, hw, pos_rand, interpret=False):
    whole = lambda x: pl.BlockSpec(x.shape, lambda i: (0,) * x.ndim)
    args = (all_emb, w1, b1.reshape(1, 64), w2, b2.reshape(1, 1), dw,
            hw.reshape(N, 1), pos_rand.reshape(N, 1))
    loss_sum, reg = pl.pallas_call(
        _loss_kernel,
        grid=(GRID,),
        in_specs=[whole(a) for a in args],
        out_specs=[pl.BlockSpec((1, 1), lambda i: (0, 0))] * 2,
        out_shape=[jax.ShapeDtypeStruct((1, 1), jnp.float32)] * 2,
        interpret=interpret,
    )(*args)
    return loss_sum[0, 0] / N + ALPHA * reg[0, 0]


def kernel(emb_vision, emb_nlp, emb_security, emb_medical, hard_sample_weights,
           W1, b1, W2, b2, domain_weights, domain_ids, pos_rand):
    all_emb = jnp.concatenate([emb_vision, emb_nlp, emb_security, emb_medical],
                              axis=0)
    return _run(all_emb, W1, b1, W2, b2, domain_weights, hard_sample_weights,
                pos_rand)
